# baseline (speedup 1.0000x reference)
import functools

import jax
import jax.numpy as jnp
from jax import lax
from jax.experimental import pallas as pl
from jax.experimental.pallas import tpu as pltpu
from jax.experimental.pallas import tpu_sc as plsc


def _gating_body(x_ref, wg_ref, bg_ref, w1_ref,
                 probs_ref, idx_ref, gates_ref, lrank_ref, bcount_ref,
                 xpk_ref, w1b_ref):
    w1b_ref[...] = w1_ref[...].astype(jnp.bfloat16)
    x = x_ref[...]
    D2 = x.shape[1] // 2
    l16 = lax.bitcast_convert_type(x[:, :D2].astype(jnp.bfloat16), jnp.uint16)
    h16 = lax.bitcast_convert_type(x[:, D2:].astype(jnp.bfloat16), jnp.uint16)
    w = l16.astype(jnp.uint32) | (h16.astype(jnp.uint32) << 16)
    xpk_ref[...] = lax.bitcast_convert_type(w, jnp.float32)
    logits = jnp.dot(x, wg_ref[...], preferred_element_type=jnp.float32)
    logits = logits + bg_ref[...]
    BT, E = logits.shape
    m1 = jnp.max(logits, axis=1, keepdims=True)
    i1 = jnp.argmax(logits, axis=1)
    col = lax.broadcasted_iota(jnp.int32, (BT, E), 1)
    neg_inf = jnp.float32(-jnp.inf)
    masked = jnp.where(col == i1[:, None], neg_inf, logits)
    m2 = jnp.max(masked, axis=1, keepdims=True)
    i2 = jnp.argmax(masked, axis=1)

    ex = jnp.exp(logits - m1)
    probs_ref[...] = ex / jnp.sum(ex, axis=1, keepdims=True)
    idx_ref[...] = jnp.stack([i1, i2], axis=1)

    e2 = jnp.exp(m2 - m1)
    g1 = 1.0 / (1.0 + e2)
    g2 = e2 / (1.0 + e2)
    gates_ref[...] = jnp.concatenate([g1, g2], axis=1)

    oh0 = (col == i1[:, None]).astype(jnp.float32)
    oh1 = (col == i2[:, None]).astype(jnp.float32)
    row = lax.broadcasted_iota(jnp.int32, (BT, BT), 0)
    colt = lax.broadcasted_iota(jnp.int32, (BT, BT), 1)
    tril = (row > colt).astype(jnp.float32)
    s = jax.lax.dot(tril, oh0 + oh1, precision=jax.lax.Precision.HIGHEST,
                    preferred_element_type=jnp.float32)
    r0 = jnp.sum(s * oh0, axis=1, keepdims=True)
    r1 = jnp.sum(s * oh1, axis=1, keepdims=True)
    lrank_ref[...] = jnp.concatenate([r0, r1], axis=1).astype(jnp.int32)
    bcount_ref[...] = jnp.sum(oh0 + oh1, axis=0).astype(jnp.int32).reshape(1, 1, E)


def _make_meta_body(num_tb, bt, blk, nb):
    def _meta_body(bc_ref, idx_ref, lrank_ref, pos_ref, bexp_ref):
        T = num_tb
        bc = bc_ref[...].reshape(T, -1).astype(jnp.float32)
        E = bc.shape[1]
        rt = lax.broadcasted_iota(jnp.int32, (T, T), 0)
        ct = lax.broadcasted_iota(jnp.int32, (T, T), 1)
        trilT = (rt > ct).astype(jnp.float32)
        rank_base = jnp.sum(trilT[:, :, None] * bc[None, :, :], axis=1)
        count = jnp.sum(bc, axis=0, keepdims=True)
        padded = jnp.ceil(count / blk) * blk
        re = lax.broadcasted_iota(jnp.int32, (E, E), 0)
        ce = lax.broadcasted_iota(jnp.int32, (E, E), 1)
        ue = (re <= ce).astype(jnp.float32)
        cum_incl = jnp.sum(padded[0, :, None] * ue, axis=0, keepdims=True)
        seg_start = cum_incl - padded
        base_te = seg_start + rank_base

        idx = idx_ref[...]
        lrank = lrank_ref[...]
        B = idx.shape[0]
        btok = lax.broadcasted_iota(jnp.int32, (B, 1), 0) // bt
        ohtb = (btok == lax.broadcasted_iota(jnp.int32, (B, T), 1)).astype(jnp.float32)
        base_full = jnp.sum(ohtb[:, :, None] * base_te[None, :, :], axis=1)
        colE = lax.broadcasted_iota(jnp.int32, (B, E), 1)
        p0 = jnp.sum(jnp.where(colE == idx[:, 0:1], base_full, 0.0), axis=1,
                     keepdims=True)
        p1 = jnp.sum(jnp.where(colE == idx[:, 1:2], base_full, 0.0), axis=1,
                     keepdims=True)
        pos_ref[...] = jnp.concatenate([p0, p1], axis=1).astype(jnp.int32) + lrank

        rs = (lax.broadcasted_iota(jnp.int32, (nb, 1), 0) * blk).astype(jnp.float32)
        bexp = jnp.sum((cum_incl <= rs).astype(jnp.float32), axis=1, keepdims=True)
        bexp_ref[...] = jnp.minimum(bexp, E - 1).astype(jnp.int32)
    return _meta_body


def _wconv_body(w2_ref, w2b_ref):
    w2b_ref[...] = w2_ref[...].astype(jnp.bfloat16)


def _gffn_body(bexp_ref, xs_ref, w1_ref, b1_ref, w2_ref, b2_ref, ys_ref):
    del bexp_ref
    w = lax.bitcast_convert_type(xs_ref[...], jnp.uint32)
    D2 = w.shape[1]
    lo = lax.bitcast_convert_type(w << 16, jnp.float32)
    hi = lax.bitcast_convert_type(w & jnp.uint32(0xFFFF0000), jnp.float32)
    lo = lo.astype(jnp.bfloat16)
    hi = hi.astype(jnp.bfloat16)
    h = (jnp.dot(lo, w1_ref[0, :D2, :], preferred_element_type=jnp.float32)
         + jnp.dot(hi, w1_ref[0, D2:, :], preferred_element_type=jnp.float32)
         + b1_ref[0])
    h = jnp.maximum(h, 0.0).astype(jnp.bfloat16)
    y = jnp.dot(h, w2_ref[0], preferred_element_type=jnp.float32) + b2_ref[0]
    l16 = lax.bitcast_convert_type(y[:, :D2].astype(jnp.bfloat16), jnp.uint16)
    h16 = lax.bitcast_convert_type(y[:, D2:].astype(jnp.bfloat16), jnp.uint16)
    wo = l16.astype(jnp.uint32) | (h16.astype(jnp.uint32) << 16)
    ys_ref[...] = lax.bitcast_convert_type(wo, jnp.float32)


def _unpack_add_body(y0_ref, y1_ref, gates_ref, out_ref):
    w0 = lax.bitcast_convert_type(y0_ref[...], jnp.uint32)
    w1 = lax.bitcast_convert_type(y1_ref[...], jnp.uint32)
    msk = jnp.uint32(0xFFFF0000)
    g0 = gates_ref[:, 0:1]
    g1 = gates_ref[:, 1:2]
    lo = (g0 * lax.bitcast_convert_type(w0 << 16, jnp.float32)
          + g1 * lax.bitcast_convert_type(w1 << 16, jnp.float32))
    hi = (g0 * lax.bitcast_convert_type(w0 & msk, jnp.float32)
          + g1 * lax.bitcast_convert_type(w1 & msk, jnp.float32))
    out_ref[...] = jnp.concatenate([lo, hi], axis=1)


def kernel(x, Wg, bg, W1, b1, W2, b2):
    B, D = x.shape
    E = Wg.shape[1]
    H = W1.shape[2]
    BT = B // E
    num_tb = B // BT
    BLK = 128
    NB = (2 * B) // BLK + E
    NPAD = NB * BLK

    probs, idx, gates, lrank, bcount, xpk, W1b = pl.pallas_call(
        _gating_body,
        grid=(num_tb,),
        in_specs=[
            pl.BlockSpec((BT, D), lambda t: (t, 0)),
            pl.BlockSpec((D, E), lambda t: (0, 0)),
            pl.BlockSpec((1, E), lambda t: (0, 0)),
            pl.BlockSpec((1, D, H), lambda t: (t, 0, 0)),
        ],
        out_specs=[
            pl.BlockSpec((BT, E), lambda t: (t, 0)),
            pl.BlockSpec((BT, 2), lambda t: (t, 0)),
            pl.BlockSpec((BT, 2), lambda t: (t, 0)),
            pl.BlockSpec((BT, 2), lambda t: (t, 0)),
            pl.BlockSpec((1, 1, E), lambda t: (t, 0, 0)),
            pl.BlockSpec((BT, D // 2), lambda t: (t, 0)),
            pl.BlockSpec((1, D, H), lambda t: (t, 0, 0)),
        ],
        out_shape=[
            jax.ShapeDtypeStruct((B, E), jnp.float32),
            jax.ShapeDtypeStruct((B, 2), jnp.int32),
            jax.ShapeDtypeStruct((B, 2), jnp.float32),
            jax.ShapeDtypeStruct((B, 2), jnp.int32),
            jax.ShapeDtypeStruct((num_tb, 1, E), jnp.int32),
            jax.ShapeDtypeStruct((B, D // 2), jnp.float32),
            jax.ShapeDtypeStruct((E, D, H), jnp.bfloat16),
        ],
    )(x, Wg, bg.reshape(1, E), W1)

    W2b = pl.pallas_call(
        _wconv_body,
        grid=(E,),
        in_specs=[
            pl.BlockSpec((1, H, D), lambda e: (e, 0, 0)),
        ],
        out_specs=pl.BlockSpec((1, H, D), lambda e: (e, 0, 0)),
        out_shape=jax.ShapeDtypeStruct((E, H, D), jnp.bfloat16),
    )(W2)

    pos, bexp = pl.pallas_call(
        _make_meta_body(num_tb, BT, BLK, NB),
        out_shape=[
            jax.ShapeDtypeStruct((B, 2), jnp.int32),
            jax.ShapeDtypeStruct((NB, 1), jnp.int32),
        ],
    )(bcount, idx, lrank)

    p0 = pos[:, 0]
    p1 = pos[:, 1]
    bexp_flat = bexp.reshape(NB)

    info = plsc.get_sparse_core_info()
    NC, NS = info.num_cores, info.num_subcores
    NW = NC * NS
    tok_w = B // NW
    mesh = plsc.VectorSubcoreMesh(core_axis_name="c", subcore_axis_name="s")

    SUB = min(64, tok_w)

    @functools.partial(
        pl.kernel, mesh=mesh,
        out_type=jax.ShapeDtypeStruct((NPAD, D // 2), jnp.float32),
        scratch_types=[
            pltpu.VMEM((SUB, D // 2), jnp.float32),
            pltpu.VMEM((SUB,), jnp.int32),
            pltpu.VMEM((SUB,), jnp.int32),
            pltpu.SemaphoreType.DMA,
        ],
    )
    def _sc_scatter(x_hbm, p0_hbm, p1_hbm,
                    xs_hbm, xbuf, i0, i1, sem):
        wid = lax.axis_index("s") * NC + lax.axis_index("c")
        for sC in range(tok_w // SUB):
            rb = wid * tok_w + sC * SUB
            lds = [
                pltpu.async_copy(p0_hbm.at[pl.ds(rb, SUB)], i0, sem),
                pltpu.async_copy(p1_hbm.at[pl.ds(rb, SUB)], i1, sem),
                pltpu.async_copy(x_hbm.at[pl.ds(rb, SUB)], xbuf, sem),
            ]
            for c in lds:
                c.wait()
            sts = [
                pltpu.async_copy(xbuf, xs_hbm.at[i0], sem),
                pltpu.async_copy(xbuf, xs_hbm.at[i1], sem),
            ]
            for c in sts:
                c.wait()

    xs = _sc_scatter(xpk, p0, p1)

    ys = pl.pallas_call(
        _gffn_body,
        grid_spec=pltpu.PrefetchScalarGridSpec(
            num_scalar_prefetch=1,
            grid=(NB,),
            in_specs=[
                pl.BlockSpec((BLK, D // 2), lambda i, be: (i, 0)),
                pl.BlockSpec((1, D, H), lambda i, be: (be[i], 0, 0)),
                pl.BlockSpec((1, 1, H), lambda i, be: (be[i], 0, 0)),
                pl.BlockSpec((1, H, D), lambda i, be: (be[i], 0, 0)),
                pl.BlockSpec((1, 1, D), lambda i, be: (be[i], 0, 0)),
            ],
            out_specs=pl.BlockSpec((BLK, D // 2), lambda i, be: (i, 0)),
        ),
        out_shape=jax.ShapeDtypeStruct((NPAD, D // 2), jnp.float32),
    )(bexp_flat, xs, W1b, b1.reshape(E, 1, H), W2b, b2.reshape(E, 1, D))

    CSUB = min(64, tok_w)

    @functools.partial(
        pl.kernel, mesh=mesh,
        out_type=[
            jax.ShapeDtypeStruct((B, D // 2), jnp.float32),
            jax.ShapeDtypeStruct((B, D // 2), jnp.float32),
        ],
        scratch_types=[
            pltpu.VMEM((CSUB, D // 2), jnp.float32),
            pltpu.VMEM((CSUB, D // 2), jnp.float32),
            pltpu.VMEM((CSUB,), jnp.int32),
            pltpu.VMEM((CSUB,), jnp.int32),
            pltpu.SemaphoreType.DMA,
        ],
    )
    def _sc_combine(ys_hbm, p0_hbm, p1_hbm, yg0_hbm, yg1_hbm,
                    y0, y1, i0, i1, sem):
        wid = lax.axis_index("s") * NC + lax.axis_index("c")
        for sC in range(tok_w // CSUB):
            rb = wid * tok_w + sC * CSUB
            ca = pltpu.async_copy(p0_hbm.at[pl.ds(rb, CSUB)], i0, sem)
            cb = pltpu.async_copy(p1_hbm.at[pl.ds(rb, CSUB)], i1, sem)
            ca.wait()
            cb.wait()
            cp0 = pltpu.async_copy(ys_hbm.at[i0], y0, sem)
            cp1 = pltpu.async_copy(ys_hbm.at[i1], y1, sem)
            cp0.wait()
            cp1.wait()
            co0 = pltpu.async_copy(y0, yg0_hbm.at[pl.ds(rb, CSUB)], sem)
            co1 = pltpu.async_copy(y1, yg1_hbm.at[pl.ds(rb, CSUB)], sem)
            co0.wait()
            co1.wait()

    yg0, yg1 = _sc_combine(ys, p0, p1)

    BT2 = min(512, B)
    out = pl.pallas_call(
        _unpack_add_body,
        grid=(B // BT2,),
        in_specs=[
            pl.BlockSpec((BT2, D // 2), lambda t: (t, 0)),
            pl.BlockSpec((BT2, D // 2), lambda t: (t, 0)),
            pl.BlockSpec((BT2, 2), lambda t: (t, 0)),
        ],
        out_specs=pl.BlockSpec((BT2, D), lambda t: (t, 0)),
        out_shape=jax.ShapeDtypeStruct((B, D), jnp.float32),
    )(yg0, yg1, gates)
    return out, probs, idx


# speedup vs baseline: 1.1434x; 1.1434x over previous
import functools

import jax
import jax.numpy as jnp
from jax import lax
from jax.experimental import pallas as pl
from jax.experimental.pallas import tpu as pltpu
from jax.experimental.pallas import tpu_sc as plsc


def _gating_body(x_ref, wg_ref, bg_ref, w1_ref,
                 probs_ref, idx_ref, gates_ref, lrank_ref, idxt_ref,
                 bcount_ref, xpk_ref, w1b_ref):
    w1b_ref[...] = w1_ref[...].astype(jnp.bfloat16)
    x = x_ref[...]
    D2 = x.shape[1] // 2
    l16 = lax.bitcast_convert_type(x[:, :D2].astype(jnp.bfloat16), jnp.uint16)
    h16 = lax.bitcast_convert_type(x[:, D2:].astype(jnp.bfloat16), jnp.uint16)
    w = l16.astype(jnp.uint32) | (h16.astype(jnp.uint32) << 16)
    xpk_ref[...] = lax.bitcast_convert_type(w, jnp.float32)
    logits = jnp.dot(x, wg_ref[...], preferred_element_type=jnp.float32)
    logits = logits + bg_ref[...]
    BT, E = logits.shape
    m1 = jnp.max(logits, axis=1, keepdims=True)
    i1 = jnp.argmax(logits, axis=1)
    col = lax.broadcasted_iota(jnp.int32, (BT, E), 1)
    neg_inf = jnp.float32(-jnp.inf)
    masked = jnp.where(col == i1[:, None], neg_inf, logits)
    m2 = jnp.max(masked, axis=1, keepdims=True)
    i2 = jnp.argmax(masked, axis=1)

    ex = jnp.exp(logits - m1)
    probs_ref[...] = ex / jnp.sum(ex, axis=1, keepdims=True)
    idx_ref[...] = jnp.stack([i1, i2], axis=1)

    e2 = jnp.exp(m2 - m1)
    g1 = 1.0 / (1.0 + e2)
    g2 = e2 / (1.0 + e2)
    gates_ref[...] = jnp.concatenate([g1, g2], axis=1)

    oh0 = (col == i1[:, None]).astype(jnp.float32)
    oh1 = (col == i2[:, None]).astype(jnp.float32)
    row = lax.broadcasted_iota(jnp.int32, (BT, BT), 0)
    colt = lax.broadcasted_iota(jnp.int32, (BT, BT), 1)
    tril = (row > colt).astype(jnp.float32)
    s = jax.lax.dot(tril, oh0 + oh1, precision=jax.lax.Precision.HIGHEST,
                    preferred_element_type=jnp.float32)
    r0 = jnp.sum(s * oh0, axis=1, keepdims=True)
    r1 = jnp.sum(s * oh1, axis=1, keepdims=True)
    lrank_ref[...] = jnp.concatenate([r0, r1], axis=1).astype(jnp.int32).T
    idxt_ref[...] = jnp.stack([i1, i2], axis=0)
    bcount_ref[...] = jnp.sum(oh0 + oh1, axis=0).astype(jnp.int32).reshape(1, 1, E)


def _make_meta_body(num_tb, bt, blk, nb):
    def _meta_body(bc_ref, idxt_ref, lrank_ref, pos_ref, bexp_ref):
        T = num_tb
        bc = bc_ref[...].reshape(T, -1).astype(jnp.float32)
        E = bc.shape[1]
        rt = lax.broadcasted_iota(jnp.int32, (T, T), 0)
        ct = lax.broadcasted_iota(jnp.int32, (T, T), 1)
        trilT = (rt > ct).astype(jnp.float32)
        rank_base = jnp.sum(trilT[:, :, None] * bc[None, :, :], axis=1)
        count = jnp.sum(bc, axis=0, keepdims=True)
        padded = jnp.ceil(count / blk) * blk
        re = lax.broadcasted_iota(jnp.int32, (E, E), 0)
        ce = lax.broadcasted_iota(jnp.int32, (E, E), 1)
        ue = (re <= ce).astype(jnp.float32)
        cum_incl = jnp.sum(padded[0, :, None] * ue, axis=0, keepdims=True)
        seg_start = cum_incl - padded
        base_et = (seg_start + rank_base).astype(jnp.int32).T

        idxt = idxt_ref[...]
        lrank = lrank_ref[...]
        B = idxt.shape[1]
        tb = lax.broadcasted_iota(jnp.int32, (E, B), 1) // bt
        base_full = jnp.zeros((E, B), jnp.int32)
        for t in range(T):
            base_full = jnp.where(tb == t, base_et[:, t:t + 1], base_full)
        erow = lax.broadcasted_iota(jnp.int32, (E, B), 0)
        p0 = jnp.sum(jnp.where(erow == idxt[0:1, :], base_full, 0), axis=0,
                     keepdims=True)
        p1 = jnp.sum(jnp.where(erow == idxt[1:2, :], base_full, 0), axis=0,
                     keepdims=True)
        pos_ref[...] = jnp.concatenate([p0, p1], axis=0) + lrank

        rs = (lax.broadcasted_iota(jnp.int32, (nb, 1), 0) * blk).astype(jnp.float32)
        bexp = jnp.sum((cum_incl <= rs).astype(jnp.float32), axis=1, keepdims=True)
        bexp_ref[...] = jnp.minimum(bexp, E - 1).astype(jnp.int32)
    return _meta_body


def _wconv_body(w2_ref, w2b_ref):
    w2b_ref[...] = w2_ref[...].astype(jnp.bfloat16)


def _gffn_body(bexp_ref, xs_ref, w1_ref, b1_ref, w2_ref, b2_ref, ys_ref):
    del bexp_ref
    w = lax.bitcast_convert_type(xs_ref[...], jnp.uint32)
    D2 = w.shape[1]
    lo = lax.bitcast_convert_type(w << 16, jnp.float32)
    hi = lax.bitcast_convert_type(w & jnp.uint32(0xFFFF0000), jnp.float32)
    lo = lo.astype(jnp.bfloat16)
    hi = hi.astype(jnp.bfloat16)
    h = (jnp.dot(lo, w1_ref[0, :D2, :], preferred_element_type=jnp.float32)
         + jnp.dot(hi, w1_ref[0, D2:, :], preferred_element_type=jnp.float32)
         + b1_ref[0])
    h = jnp.maximum(h, 0.0).astype(jnp.bfloat16)
    y = jnp.dot(h, w2_ref[0], preferred_element_type=jnp.float32) + b2_ref[0]
    l16 = lax.bitcast_convert_type(y[:, :D2].astype(jnp.bfloat16), jnp.uint16)
    h16 = lax.bitcast_convert_type(y[:, D2:].astype(jnp.bfloat16), jnp.uint16)
    wo = l16.astype(jnp.uint32) | (h16.astype(jnp.uint32) << 16)
    ys_ref[...] = lax.bitcast_convert_type(wo, jnp.float32)


def _unpack_add_body(y0_ref, y1_ref, gates_ref, out_ref):
    w0 = lax.bitcast_convert_type(y0_ref[...], jnp.uint32)
    w1 = lax.bitcast_convert_type(y1_ref[...], jnp.uint32)
    msk = jnp.uint32(0xFFFF0000)
    g0 = gates_ref[:, 0:1]
    g1 = gates_ref[:, 1:2]
    lo = (g0 * lax.bitcast_convert_type(w0 << 16, jnp.float32)
          + g1 * lax.bitcast_convert_type(w1 << 16, jnp.float32))
    hi = (g0 * lax.bitcast_convert_type(w0 & msk, jnp.float32)
          + g1 * lax.bitcast_convert_type(w1 & msk, jnp.float32))
    out_ref[...] = jnp.concatenate([lo, hi], axis=1)


def kernel(x, Wg, bg, W1, b1, W2, b2):
    B, D = x.shape
    E = Wg.shape[1]
    H = W1.shape[2]
    BT = B // E
    num_tb = B // BT
    BLK = 256
    NB = (2 * B) // BLK + E
    NPAD = NB * BLK

    probs, idx, gates, lrank, idxt, bcount, xpk, W1b = pl.pallas_call(
        _gating_body,
        grid=(num_tb,),
        in_specs=[
            pl.BlockSpec((BT, D), lambda t: (t, 0)),
            pl.BlockSpec((D, E), lambda t: (0, 0)),
            pl.BlockSpec((1, E), lambda t: (0, 0)),
            pl.BlockSpec((1, D, H), lambda t: (t, 0, 0)),
        ],
        out_specs=[
            pl.BlockSpec((BT, E), lambda t: (t, 0)),
            pl.BlockSpec((BT, 2), lambda t: (t, 0)),
            pl.BlockSpec((BT, 2), lambda t: (t, 0)),
            pl.BlockSpec((2, BT), lambda t: (0, t)),
            pl.BlockSpec((2, BT), lambda t: (0, t)),
            pl.BlockSpec((1, 1, E), lambda t: (t, 0, 0)),
            pl.BlockSpec((BT, D // 2), lambda t: (t, 0)),
            pl.BlockSpec((1, D, H), lambda t: (t, 0, 0)),
        ],
        out_shape=[
            jax.ShapeDtypeStruct((B, E), jnp.float32),
            jax.ShapeDtypeStruct((B, 2), jnp.int32),
            jax.ShapeDtypeStruct((B, 2), jnp.float32),
            jax.ShapeDtypeStruct((2, B), jnp.int32),
            jax.ShapeDtypeStruct((2, B), jnp.int32),
            jax.ShapeDtypeStruct((num_tb, 1, E), jnp.int32),
            jax.ShapeDtypeStruct((B, D // 2), jnp.float32),
            jax.ShapeDtypeStruct((E, D, H), jnp.bfloat16),
        ],
    )(x, Wg, bg.reshape(1, E), W1)

    W2b = pl.pallas_call(
        _wconv_body,
        grid=(E,),
        in_specs=[
            pl.BlockSpec((1, H, D), lambda e: (e, 0, 0)),
        ],
        out_specs=pl.BlockSpec((1, H, D), lambda e: (e, 0, 0)),
        out_shape=jax.ShapeDtypeStruct((E, H, D), jnp.bfloat16),
    )(W2)

    pos, bexp = pl.pallas_call(
        _make_meta_body(num_tb, BT, BLK, NB),
        out_shape=[
            jax.ShapeDtypeStruct((2, B), jnp.int32),
            jax.ShapeDtypeStruct((NB, 1), jnp.int32),
        ],
    )(bcount, idxt, lrank)

    p0 = pos[0]
    p1 = pos[1]
    bexp_flat = bexp.reshape(NB)

    info = plsc.get_sparse_core_info()
    NC, NS = info.num_cores, info.num_subcores
    NW = NC * NS
    tok_w = B // NW
    mesh = plsc.VectorSubcoreMesh(core_axis_name="c", subcore_axis_name="s")

    SUB = min(64, tok_w)

    @functools.partial(
        pl.kernel, mesh=mesh,
        out_type=jax.ShapeDtypeStruct((NPAD, D // 2), jnp.float32),
        scratch_types=[
            pltpu.VMEM((SUB, D // 2), jnp.float32),
            pltpu.VMEM((SUB,), jnp.int32),
            pltpu.VMEM((SUB,), jnp.int32),
            pltpu.SemaphoreType.DMA,
        ],
    )
    def _sc_scatter(x_hbm, p0_hbm, p1_hbm,
                    xs_hbm, xbuf, i0, i1, sem):
        wid = lax.axis_index("s") * NC + lax.axis_index("c")
        for sC in range(tok_w // SUB):
            rb = wid * tok_w + sC * SUB
            lds = [
                pltpu.async_copy(p0_hbm.at[pl.ds(rb, SUB)], i0, sem),
                pltpu.async_copy(p1_hbm.at[pl.ds(rb, SUB)], i1, sem),
                pltpu.async_copy(x_hbm.at[pl.ds(rb, SUB)], xbuf, sem),
            ]
            for c in lds:
                c.wait()
            sts = [
                pltpu.async_copy(xbuf, xs_hbm.at[i0], sem),
                pltpu.async_copy(xbuf, xs_hbm.at[i1], sem),
            ]
            for c in sts:
                c.wait()

    xs = _sc_scatter(xpk, p0, p1)

    ys = pl.pallas_call(
        _gffn_body,
        grid_spec=pltpu.PrefetchScalarGridSpec(
            num_scalar_prefetch=1,
            grid=(NB,),
            in_specs=[
                pl.BlockSpec((BLK, D // 2), lambda i, be: (i, 0)),
                pl.BlockSpec((1, D, H), lambda i, be: (be[i], 0, 0)),
                pl.BlockSpec((1, 1, H), lambda i, be: (be[i], 0, 0)),
                pl.BlockSpec((1, H, D), lambda i, be: (be[i], 0, 0)),
                pl.BlockSpec((1, 1, D), lambda i, be: (be[i], 0, 0)),
            ],
            out_specs=pl.BlockSpec((BLK, D // 2), lambda i, be: (i, 0)),
        ),
        out_shape=jax.ShapeDtypeStruct((NPAD, D // 2), jnp.float32),
    )(bexp_flat, xs, W1b, b1.reshape(E, 1, H), W2b, b2.reshape(E, 1, D))

    CSUB = min(64, tok_w)

    @functools.partial(
        pl.kernel, mesh=mesh,
        out_type=[
            jax.ShapeDtypeStruct((B, D // 2), jnp.float32),
            jax.ShapeDtypeStruct((B, D // 2), jnp.float32),
        ],
        scratch_types=[
            pltpu.VMEM((CSUB, D // 2), jnp.float32),
            pltpu.VMEM((CSUB, D // 2), jnp.float32),
            pltpu.VMEM((CSUB,), jnp.int32),
            pltpu.VMEM((CSUB,), jnp.int32),
            pltpu.SemaphoreType.DMA,
        ],
    )
    def _sc_combine(ys_hbm, p0_hbm, p1_hbm, yg0_hbm, yg1_hbm,
                    y0, y1, i0, i1, sem):
        wid = lax.axis_index("s") * NC + lax.axis_index("c")
        for sC in range(tok_w // CSUB):
            rb = wid * tok_w + sC * CSUB
            ca = pltpu.async_copy(p0_hbm.at[pl.ds(rb, CSUB)], i0, sem)
            cb = pltpu.async_copy(p1_hbm.at[pl.ds(rb, CSUB)], i1, sem)
            ca.wait()
            cb.wait()
            cp0 = pltpu.async_copy(ys_hbm.at[i0], y0, sem)
            cp1 = pltpu.async_copy(ys_hbm.at[i1], y1, sem)
            cp0.wait()
            cp1.wait()
            co0 = pltpu.async_copy(y0, yg0_hbm.at[pl.ds(rb, CSUB)], sem)
            co1 = pltpu.async_copy(y1, yg1_hbm.at[pl.ds(rb, CSUB)], sem)
            co0.wait()
            co1.wait()

    yg0, yg1 = _sc_combine(ys, p0, p1)

    BT2 = min(512, B)
    out = pl.pallas_call(
        _unpack_add_body,
        grid=(B // BT2,),
        in_specs=[
            pl.BlockSpec((BT2, D // 2), lambda t: (t, 0)),
            pl.BlockSpec((BT2, D // 2), lambda t: (t, 0)),
            pl.BlockSpec((BT2, 2), lambda t: (t, 0)),
        ],
        out_specs=pl.BlockSpec((BT2, D), lambda t: (t, 0)),
        out_shape=jax.ShapeDtypeStruct((B, D), jnp.float32),
    )(yg0, yg1, gates)
    return out, probs, idx
